# Initial kernel scaffold; baseline (speedup 1.0000x reference)
import jax
import jax.numpy as jnp
from jax.experimental import pallas as pl


def kernel(feats, coords, params):
    raise NotImplementedError("write your pallas kernel here")


# speedup vs baseline: 16.0094x; 16.0094x over previous
import jax
import jax.numpy as jnp
from jax.experimental import pallas as pl

VOXEL = 0.005
BN_EPS = 1e-5
N_POINTS = 32768
MAX_INPUT = 1024

_HI = jax.lax.Precision.HIGHEST


def _dot(a, b):
    return jax.lax.dot_general(a, b, (((1,), (0,)), ((), ())),
                               precision=_HI,
                               preferred_element_type=jnp.float32)


def _iota(shape, dim, dtype=jnp.float32):
    return jax.lax.broadcasted_iota(jnp.int32, shape, dim).astype(dtype)


def _fps_big_body(x_ref, y_ref, z_ref, a_ref, b_ref, c_ref, out_ref):
    X = x_ref[...] * VOXEL
    Y = y_ref[...] * VOXEL
    Z = z_ref[...] * VOXEL
    A = a_ref[...]
    B = b_ref[...]
    C = c_ref[...]
    flat = _iota(X.shape, 0) * 128.0 + _iota(X.shape, 1)
    lane8 = _iota((1, 8), 1)

    def body(i, carry):
        dists, fidx = carry
        oh = flat == fidx
        px = jnp.sum(jnp.where(oh, X, 0.0))
        py = jnp.sum(jnp.where(oh, Y, 0.0))
        pz = jnp.sum(jnp.where(oh, Z, 0.0))
        pa = jnp.sum(jnp.where(oh, A, 0.0))
        pb = jnp.sum(jnp.where(oh, B, 0.0))
        pc = jnp.sum(jnp.where(oh, C, 0.0))
        rv = (jnp.where(lane8 == 0.0, px, 0.0)
              + jnp.where(lane8 == 1.0, py, 0.0)
              + jnp.where(lane8 == 2.0, pz, 0.0)
              + jnp.where(lane8 == 3.0, pa, 0.0)
              + jnp.where(lane8 == 4.0, pb, 0.0)
              + jnp.where(lane8 == 5.0, pc, 0.0))
        out_ref[pl.ds(i, 1), :] = rv
        dx = X - px
        dy = Y - py
        dz = Z - pz
        d = dx * dx + dy * dy + dz * dz
        dists = jnp.minimum(dists, d)
        m = jnp.max(dists)
        fidx = jnp.min(jnp.where(dists == m, flat, 1e9))
        return dists, fidx

    dists0 = jnp.full(X.shape, 1e10, dtype=jnp.float32)
    jax.lax.fori_loop(0, MAX_INPUT, body, (dists0, 0.0))


def _fps_big(cx, feats):
    ins = [cx[:, 0].reshape(256, 128), cx[:, 1].reshape(256, 128),
           cx[:, 2].reshape(256, 128),
           feats[:, 0].reshape(256, 128), feats[:, 1].reshape(256, 128),
           feats[:, 2].reshape(256, 128)]
    out = pl.pallas_call(
        _fps_big_body,
        out_shape=jax.ShapeDtypeStruct((MAX_INPUT, 8), jnp.float32),
    )(*ins)
    return out


def _make_sa_body(P, Np, radius, nsample):
    r2 = radius * radius

    def body(kxyz_ref, kxyzT_ref, kfeats_ref, w1_ref, b1_ref, w2_ref, b2_ref,
             nxyz_ref, nfeat_ref):
        Xr = kxyzT_ref[0:1, :]
        Yr = kxyzT_ref[1:2, :]
        Zr = kxyzT_ref[2:3, :]
        lane = _iota((1, Np), 1)
        lane3 = _iota((1, 3), 1)

        def fps_body(i, carry):
            dists, fidx = carry
            oh = lane == fidx
            px = jnp.sum(jnp.where(oh, Xr, 0.0))
            py = jnp.sum(jnp.where(oh, Yr, 0.0))
            pz = jnp.sum(jnp.where(oh, Zr, 0.0))
            rv = (jnp.where(lane3 == 0.0, px, 0.0)
                  + jnp.where(lane3 == 1.0, py, 0.0)
                  + jnp.where(lane3 == 2.0, pz, 0.0))
            nxyz_ref[pl.ds(i, 1), :] = rv
            dx = Xr - px
            dy = Yr - py
            dz = Zr - pz
            d = dx * dx + dy * dy + dz * dz
            dists = jnp.minimum(dists, d)
            m = jnp.max(dists)
            fidx = jnp.min(jnp.where(dists == m, lane, 1e9))
            return dists, fidx

        dists0 = jnp.full((1, Np), 1e10, dtype=jnp.float32)
        jax.lax.fori_loop(0, P, fps_body, (dists0, 0.0))

        u = nxyz_ref[...]
        ux = u[:, 0:1]
        uy = u[:, 1:2]
        uz = u[:, 2:3]
        dx = ux - Xr
        dy = uy - Yr
        dz = uz - Zr
        d2 = dx * dx + dy * dy + dz * dz
        maskf = (d2 < r2).astype(jnp.float32)
        lt = (_iota((Np, Np), 0) < _iota((Np, Np), 1)).astype(jnp.float32)
        rank = _dot(maskf, lt)
        kcat = jnp.concatenate([kxyz_ref[...], kfeats_ref[...]], axis=1)

        run = None
        h0 = None
        for s in range(nsample):
            hs = maskf * (rank == float(s)).astype(jnp.float32)
            if s == 0:
                h0 = hs
            else:
                found = jnp.sum(hs, axis=1, keepdims=True)
                hs = jnp.where(found > 0.0, hs, h0)
            gs = _dot(hs, kcat)
            gs = jnp.concatenate([gs[:, 0:3] - u, gs[:, 3:]], axis=1)
            xs = jnp.maximum(_dot(gs, w1_ref[...]) + b1_ref[...], 0.0)
            xs = jnp.maximum(_dot(xs, w2_ref[...]) + b2_ref[...], 0.0)
            run = xs if run is None else jnp.maximum(run, xs)
        nfeat_ref[...] = run

    return body


def _sa(kxyz, kxyzT, kfeats, w1, b1, w2, b2, P, radius, nsample=16):
    Np = kxyz.shape[0]
    H2 = w2.shape[1]
    return pl.pallas_call(
        _make_sa_body(P, Np, radius, nsample),
        out_shape=(jax.ShapeDtypeStruct((P, 3), jnp.float32),
                   jax.ShapeDtypeStruct((P, H2), jnp.float32)),
    )(kxyz, kxyzT, kfeats, w1, b1, w2, b2)


def _make_fp_body(Nu, Nk):
    def body(uxyz_ref, kxyzT_ref, kfeats_ref, ufeats_ref,
             w1_ref, b1_ref, w2_ref, b2_ref, out_ref):
        ux = uxyz_ref[:, 0:1]
        uy = uxyz_ref[:, 1:2]
        uz = uxyz_ref[:, 2:3]
        Xr = kxyzT_ref[0:1, :]
        Yr = kxyzT_ref[1:2, :]
        Zr = kxyzT_ref[2:3, :]
        dx = ux - Xr
        dy = uy - Yr
        dz = uz - Zr
        d2 = dx * dx + dy * dy + dz * dz
        lane = _iota((1, Nk), 1)
        d2w = d2
        ohs, recips = [], []
        for _ in range(3):
            mj = jnp.min(d2w, axis=1, keepdims=True)
            idxj = jnp.min(jnp.where(d2w == mj, lane, 1e9),
                           axis=1, keepdims=True)
            oh = lane == idxj
            ohs.append(oh)
            recips.append(1.0 / (mj + 1e-8))
            d2w = jnp.where(oh, 1e30, d2w)
        ssum = (recips[0] + recips[1]) + recips[2]
        wm = (jnp.where(ohs[0], recips[0] / ssum, 0.0)
              + jnp.where(ohs[1], recips[1] / ssum, 0.0)
              + jnp.where(ohs[2], recips[2] / ssum, 0.0))
        interp = _dot(wm, kfeats_ref[...])
        x = jnp.concatenate([interp, ufeats_ref[...]], axis=1)
        x = jnp.maximum(_dot(x, w1_ref[...]) + b1_ref[...], 0.0)
        x = jnp.maximum(_dot(x, w2_ref[...]) + b2_ref[...], 0.0)
        out_ref[...] = x

    return body


def _fp(uxyz, kxyzT, kfeats, ufeats, w1, b1, w2, b2):
    Nu = uxyz.shape[0]
    Nk = kxyzT.shape[1]
    H2 = w2.shape[1]
    return pl.pallas_call(
        _make_fp_body(Nu, Nk),
        out_shape=jax.ShapeDtypeStruct((Nu, H2), jnp.float32),
    )(uxyz, kxyzT, kfeats, ufeats, w1, b1, w2, b2)


def _make_final_body(BLK, Nk):
    def body(cx_ref, kxyzT_ref, kfeats_ref, wf_ref, bf_ref, out_ref):
        ux = cx_ref[:, 0:1] * VOXEL
        uy = cx_ref[:, 1:2] * VOXEL
        uz = cx_ref[:, 2:3] * VOXEL
        Xr = kxyzT_ref[0:1, :]
        Yr = kxyzT_ref[1:2, :]
        Zr = kxyzT_ref[2:3, :]
        dx = ux - Xr
        dy = uy - Yr
        dz = uz - Zr
        d2 = dx * dx + dy * dy + dz * dz
        lane = _iota((1, Nk), 1)
        d2w = d2
        ohs, recips = [], []
        for _ in range(3):
            mj = jnp.min(d2w, axis=1, keepdims=True)
            idxj = jnp.min(jnp.where(d2w == mj, lane, 1e9),
                           axis=1, keepdims=True)
            oh = lane == idxj
            ohs.append(oh)
            recips.append(1.0 / (mj + 1e-8))
            d2w = jnp.where(oh, 1e30, d2w)
        ssum = (recips[0] + recips[1]) + recips[2]
        wm = (jnp.where(ohs[0], recips[0] / ssum, 0.0)
              + jnp.where(ohs[1], recips[1] / ssum, 0.0)
              + jnp.where(ohs[2], recips[2] / ssum, 0.0))
        interp = _dot(wm, kfeats_ref[...])
        out_ref[...] = _dot(interp, wf_ref[...]) + bf_ref[...]

    return body


def _final(cx, kxyzT, kfeats, wf, bf):
    BLK = 2048
    N = cx.shape[0]
    Nk = kxyzT.shape[1]
    H = wf.shape[1]
    grid = N // BLK
    return pl.pallas_call(
        _make_final_body(BLK, Nk),
        grid=(grid,),
        in_specs=[
            pl.BlockSpec((BLK, 3), lambda i: (i, 0)),
            pl.BlockSpec((3, Nk), lambda i: (0, 0)),
            pl.BlockSpec((Nk, kfeats.shape[1]), lambda i: (0, 0)),
            pl.BlockSpec((wf.shape[0], H), lambda i: (0, 0)),
            pl.BlockSpec((1, H), lambda i: (0, 0)),
        ],
        out_specs=pl.BlockSpec((BLK, H), lambda i: (i, 0)),
        out_shape=jax.ShapeDtypeStruct((N, H), jnp.float32),
    )(cx, kxyzT, kfeats, wf, bf)


def _fold(layer):
    W, b, g, bt = layer
    s = g / jnp.sqrt(1.0 + BN_EPS)
    return (W * s[:, None]).T, (b * s + bt)[None, :]


def kernel(feats, coords, params):
    cx = coords[:, 1:4].astype(jnp.float32)
    sub = _fps_big(cx, feats)
    xyz_sub = sub[:, 0:3]
    feats_sub = sub[:, 3:6]
    sa1 = [p for l in params['sa1'] for p in _fold(l)]
    sa2 = [p for l in params['sa2'] for p in _fold(l)]
    sa3 = [p for l in params['sa3'] for p in _fold(l)]
    fp3 = [p for l in params['fp3'] for p in _fold(l)]
    fp2 = [p for l in params['fp2'] for p in _fold(l)]
    fp1 = [p for l in params['fp1'] for p in _fold(l)]
    wf, bf = _fold(params['final'])

    l1_xyz, l1_f = _sa(xyz_sub, xyz_sub.T, feats_sub, *sa1, P=256, radius=0.04)
    l2_xyz, l2_f = _sa(l1_xyz, l1_xyz.T, l1_f, *sa2, P=64, radius=0.08)
    l3_xyz, l3_f = _sa(l2_xyz, l2_xyz.T, l2_f, *sa3, P=16, radius=0.16)
    l2_f = _fp(l2_xyz, l3_xyz.T, l3_f, l2_f, *fp3)
    l1_f = _fp(l1_xyz, l2_xyz.T, l2_f, l1_f, *fp2)
    l0_f = _fp(xyz_sub, l1_xyz.T, l1_f, feats_sub, *fp1)
    return _final(cx, xyz_sub.T, l0_f, wf, bf)


# speedup vs baseline: 16.6121x; 1.0376x over previous
import jax
import jax.numpy as jnp
from jax.experimental import pallas as pl

VOXEL = 0.005
BN_EPS = 1e-5
N_POINTS = 32768
MAX_INPUT = 1024

_HI = jax.lax.Precision.HIGHEST


def _dot(a, b):
    return jax.lax.dot_general(a, b, (((1,), (0,)), ((), ())),
                               precision=_HI,
                               preferred_element_type=jnp.float32)


def _iota(shape, dim, dtype=jnp.float32):
    return jax.lax.broadcasted_iota(jnp.int32, shape, dim).astype(dtype)


def _fps_big_body(x_ref, y_ref, z_ref, a_ref, b_ref, c_ref, out_ref):
    X = x_ref[...] * VOXEL
    Y = y_ref[...] * VOXEL
    Z = z_ref[...] * VOXEL
    flat = _iota(X.shape, 0) * 128.0 + _iota(X.shape, 1)
    lane8 = _iota((1, 8), 1)
    lane128 = _iota((1, 128), 1)

    def body(i, carry):
        dists, fidx = carry
        fi = fidx.astype(jnp.int32)
        r = fi // 128
        c = (fi - r * 128).astype(jnp.float32)
        ohl = lane128 == c

        def ext(ref):
            return jnp.sum(jnp.where(ohl, ref[pl.ds(r, 1), :], 0.0))

        px = ext(x_ref) * VOXEL
        py = ext(y_ref) * VOXEL
        pz = ext(z_ref) * VOXEL
        pa = ext(a_ref)
        pb = ext(b_ref)
        pc = ext(c_ref)
        rv = (jnp.where(lane8 == 0.0, px, 0.0)
              + jnp.where(lane8 == 1.0, py, 0.0)
              + jnp.where(lane8 == 2.0, pz, 0.0)
              + jnp.where(lane8 == 3.0, pa, 0.0)
              + jnp.where(lane8 == 4.0, pb, 0.0)
              + jnp.where(lane8 == 5.0, pc, 0.0))
        out_ref[pl.ds(i, 1), :] = rv
        dx = X - px
        dy = Y - py
        dz = Z - pz
        d = dx * dx + dy * dy + dz * dz
        dists = jnp.minimum(dists, d)
        m = jnp.max(dists)
        fidx = jnp.min(jnp.where(dists == m, flat, 1e9))
        return dists, fidx

    dists0 = jnp.full(X.shape, 1e10, dtype=jnp.float32)
    jax.lax.fori_loop(0, MAX_INPUT, body, (dists0, 0.0))


def _fps_big(cx, feats):
    ins = [cx[:, 0].reshape(256, 128), cx[:, 1].reshape(256, 128),
           cx[:, 2].reshape(256, 128),
           feats[:, 0].reshape(256, 128), feats[:, 1].reshape(256, 128),
           feats[:, 2].reshape(256, 128)]
    out = pl.pallas_call(
        _fps_big_body,
        out_shape=jax.ShapeDtypeStruct((MAX_INPUT, 8), jnp.float32),
    )(*ins)
    return out


def _make_sa_body(P, Np, radius, nsample):
    r2 = radius * radius

    def body(kxyz_ref, kxyzT_ref, kfeats_ref, w1_ref, b1_ref, w2_ref, b2_ref,
             nxyz_ref, nfeat_ref):
        Xr = kxyzT_ref[0:1, :]
        Yr = kxyzT_ref[1:2, :]
        Zr = kxyzT_ref[2:3, :]
        lane = _iota((1, Np), 1)
        lane3 = _iota((1, 3), 1)

        def fps_body(i, carry):
            dists, fidx = carry
            oh = lane == fidx
            px = jnp.sum(jnp.where(oh, Xr, 0.0))
            py = jnp.sum(jnp.where(oh, Yr, 0.0))
            pz = jnp.sum(jnp.where(oh, Zr, 0.0))
            rv = (jnp.where(lane3 == 0.0, px, 0.0)
                  + jnp.where(lane3 == 1.0, py, 0.0)
                  + jnp.where(lane3 == 2.0, pz, 0.0))
            nxyz_ref[pl.ds(i, 1), :] = rv
            dx = Xr - px
            dy = Yr - py
            dz = Zr - pz
            d = dx * dx + dy * dy + dz * dz
            dists = jnp.minimum(dists, d)
            m = jnp.max(dists)
            fidx = jnp.min(jnp.where(dists == m, lane, 1e9))
            return dists, fidx

        dists0 = jnp.full((1, Np), 1e10, dtype=jnp.float32)
        jax.lax.fori_loop(0, P, fps_body, (dists0, 0.0))

        u = nxyz_ref[...]
        ux = u[:, 0:1]
        uy = u[:, 1:2]
        uz = u[:, 2:3]
        dx = ux - Xr
        dy = uy - Yr
        dz = uz - Zr
        d2 = dx * dx + dy * dy + dz * dz
        maskf = (d2 < r2).astype(jnp.float32)
        lt = (_iota((Np, Np), 0) < _iota((Np, Np), 1)).astype(jnp.float32)
        rank = _dot(maskf, lt)
        kcat = jnp.concatenate([kxyz_ref[...], kfeats_ref[...]], axis=1)

        run = None
        h0 = None
        for s in range(nsample):
            hs = maskf * (rank == float(s)).astype(jnp.float32)
            if s == 0:
                h0 = hs
            else:
                found = jnp.sum(hs, axis=1, keepdims=True)
                hs = jnp.where(found > 0.0, hs, h0)
            gs = _dot(hs, kcat)
            gs = jnp.concatenate([gs[:, 0:3] - u, gs[:, 3:]], axis=1)
            xs = jnp.maximum(_dot(gs, w1_ref[...]) + b1_ref[...], 0.0)
            xs = jnp.maximum(_dot(xs, w2_ref[...]) + b2_ref[...], 0.0)
            run = xs if run is None else jnp.maximum(run, xs)
        nfeat_ref[...] = run

    return body


def _sa(kxyz, kxyzT, kfeats, w1, b1, w2, b2, P, radius, nsample=16):
    Np = kxyz.shape[0]
    H2 = w2.shape[1]
    return pl.pallas_call(
        _make_sa_body(P, Np, radius, nsample),
        out_shape=(jax.ShapeDtypeStruct((P, 3), jnp.float32),
                   jax.ShapeDtypeStruct((P, H2), jnp.float32)),
    )(kxyz, kxyzT, kfeats, w1, b1, w2, b2)


def _make_fp_body(Nu, Nk):
    def body(uxyz_ref, kxyzT_ref, kfeats_ref, ufeats_ref,
             w1_ref, b1_ref, w2_ref, b2_ref, out_ref):
        ux = uxyz_ref[:, 0:1]
        uy = uxyz_ref[:, 1:2]
        uz = uxyz_ref[:, 2:3]
        Xr = kxyzT_ref[0:1, :]
        Yr = kxyzT_ref[1:2, :]
        Zr = kxyzT_ref[2:3, :]
        dx = ux - Xr
        dy = uy - Yr
        dz = uz - Zr
        d2 = dx * dx + dy * dy + dz * dz
        lane = _iota((1, Nk), 1)
        d2w = d2
        ohs, recips = [], []
        for _ in range(3):
            mj = jnp.min(d2w, axis=1, keepdims=True)
            idxj = jnp.min(jnp.where(d2w == mj, lane, 1e9),
                           axis=1, keepdims=True)
            oh = lane == idxj
            ohs.append(oh)
            recips.append(1.0 / (mj + 1e-8))
            d2w = jnp.where(oh, 1e30, d2w)
        ssum = (recips[0] + recips[1]) + recips[2]
        wm = (jnp.where(ohs[0], recips[0] / ssum, 0.0)
              + jnp.where(ohs[1], recips[1] / ssum, 0.0)
              + jnp.where(ohs[2], recips[2] / ssum, 0.0))
        interp = _dot(wm, kfeats_ref[...])
        x = jnp.concatenate([interp, ufeats_ref[...]], axis=1)
        x = jnp.maximum(_dot(x, w1_ref[...]) + b1_ref[...], 0.0)
        x = jnp.maximum(_dot(x, w2_ref[...]) + b2_ref[...], 0.0)
        out_ref[...] = x

    return body


def _fp(uxyz, kxyzT, kfeats, ufeats, w1, b1, w2, b2):
    Nu = uxyz.shape[0]
    Nk = kxyzT.shape[1]
    H2 = w2.shape[1]
    return pl.pallas_call(
        _make_fp_body(Nu, Nk),
        out_shape=jax.ShapeDtypeStruct((Nu, H2), jnp.float32),
    )(uxyz, kxyzT, kfeats, ufeats, w1, b1, w2, b2)


def _make_final_body(BLK, Nk):
    def body(cx_ref, kxyzT_ref, kfeats_ref, wf_ref, bf_ref, out_ref):
        ux = cx_ref[:, 0:1] * VOXEL
        uy = cx_ref[:, 1:2] * VOXEL
        uz = cx_ref[:, 2:3] * VOXEL
        Xr = kxyzT_ref[0:1, :]
        Yr = kxyzT_ref[1:2, :]
        Zr = kxyzT_ref[2:3, :]
        dx = ux - Xr
        dy = uy - Yr
        dz = uz - Zr
        d2 = dx * dx + dy * dy + dz * dz
        lane = _iota((1, Nk), 1)
        d2w = d2
        ohs, recips = [], []
        for _ in range(3):
            mj = jnp.min(d2w, axis=1, keepdims=True)
            idxj = jnp.min(jnp.where(d2w == mj, lane, 1e9),
                           axis=1, keepdims=True)
            oh = lane == idxj
            ohs.append(oh)
            recips.append(1.0 / (mj + 1e-8))
            d2w = jnp.where(oh, 1e30, d2w)
        ssum = (recips[0] + recips[1]) + recips[2]
        wm = (jnp.where(ohs[0], recips[0] / ssum, 0.0)
              + jnp.where(ohs[1], recips[1] / ssum, 0.0)
              + jnp.where(ohs[2], recips[2] / ssum, 0.0))
        interp = _dot(wm, kfeats_ref[...])
        out_ref[...] = _dot(interp, wf_ref[...]) + bf_ref[...]

    return body


def _final(cx, kxyzT, kfeats, wf, bf):
    BLK = 2048
    N = cx.shape[0]
    Nk = kxyzT.shape[1]
    H = wf.shape[1]
    grid = N // BLK
    return pl.pallas_call(
        _make_final_body(BLK, Nk),
        grid=(grid,),
        in_specs=[
            pl.BlockSpec((BLK, 3), lambda i: (i, 0)),
            pl.BlockSpec((3, Nk), lambda i: (0, 0)),
            pl.BlockSpec((Nk, kfeats.shape[1]), lambda i: (0, 0)),
            pl.BlockSpec((wf.shape[0], H), lambda i: (0, 0)),
            pl.BlockSpec((1, H), lambda i: (0, 0)),
        ],
        out_specs=pl.BlockSpec((BLK, H), lambda i: (i, 0)),
        out_shape=jax.ShapeDtypeStruct((N, H), jnp.float32),
    )(cx, kxyzT, kfeats, wf, bf)


def _fold(layer):
    W, b, g, bt = layer
    s = g / jnp.sqrt(1.0 + BN_EPS)
    return (W * s[:, None]).T, (b * s + bt)[None, :]


def kernel(feats, coords, params):
    cx = coords[:, 1:4].astype(jnp.float32)
    sub = _fps_big(cx, feats)
    xyz_sub = sub[:, 0:3]
    feats_sub = sub[:, 3:6]
    sa1 = [p for l in params['sa1'] for p in _fold(l)]
    sa2 = [p for l in params['sa2'] for p in _fold(l)]
    sa3 = [p for l in params['sa3'] for p in _fold(l)]
    fp3 = [p for l in params['fp3'] for p in _fold(l)]
    fp2 = [p for l in params['fp2'] for p in _fold(l)]
    fp1 = [p for l in params['fp1'] for p in _fold(l)]
    wf, bf = _fold(params['final'])

    l1_xyz, l1_f = _sa(xyz_sub, xyz_sub.T, feats_sub, *sa1, P=256, radius=0.04)
    l2_xyz, l2_f = _sa(l1_xyz, l1_xyz.T, l1_f, *sa2, P=64, radius=0.08)
    l3_xyz, l3_f = _sa(l2_xyz, l2_xyz.T, l2_f, *sa3, P=16, radius=0.16)
    l2_f = _fp(l2_xyz, l3_xyz.T, l3_f, l2_f, *fp3)
    l1_f = _fp(l1_xyz, l2_xyz.T, l2_f, l1_f, *fp2)
    l0_f = _fp(xyz_sub, l1_xyz.T, l1_f, feats_sub, *fp1)
    return _final(cx, xyz_sub.T, l0_f, wf, bf)


# speedup vs baseline: 21.1825x; 1.2751x over previous
import jax
import jax.numpy as jnp
from jax.experimental import pallas as pl

VOXEL = 0.005
BN_EPS = 1e-5
N_POINTS = 32768
MAX_INPUT = 1024

_HI = jax.lax.Precision.HIGHEST


def _dot(a, b, precision=jax.lax.Precision.HIGHEST):
    return jax.lax.dot_general(a, b, (((1,), (0,)), ((), ())),
                               precision=precision,
                               preferred_element_type=jnp.float32)


def _iota(shape, dim, dtype=jnp.float32):
    return jax.lax.broadcasted_iota(jnp.int32, shape, dim).astype(dtype)


def _fps_big_body(x_ref, y_ref, z_ref, a_ref, b_ref, c_ref, out_ref):
    X = x_ref[...] * VOXEL
    Y = y_ref[...] * VOXEL
    Z = z_ref[...] * VOXEL
    flat = _iota(X.shape, 0) * 128.0 + _iota(X.shape, 1)
    lane8 = _iota((1, 8), 1)
    lane128 = _iota((1, 128), 1)

    def body(i, carry):
        dists, fidx = carry
        fi = fidx.astype(jnp.int32)
        r = fi // 128
        c = (fi - r * 128).astype(jnp.float32)
        ohl = lane128 == c

        def ext(ref):
            return jnp.sum(jnp.where(ohl, ref[pl.ds(r, 1), :], 0.0))

        px = ext(x_ref) * VOXEL
        py = ext(y_ref) * VOXEL
        pz = ext(z_ref) * VOXEL
        pa = ext(a_ref)
        pb = ext(b_ref)
        pc = ext(c_ref)
        rv = (jnp.where(lane8 == 0.0, px, 0.0)
              + jnp.where(lane8 == 1.0, py, 0.0)
              + jnp.where(lane8 == 2.0, pz, 0.0)
              + jnp.where(lane8 == 3.0, pa, 0.0)
              + jnp.where(lane8 == 4.0, pb, 0.0)
              + jnp.where(lane8 == 5.0, pc, 0.0))
        out_ref[pl.ds(i, 1), :] = rv
        dx = X - px
        dy = Y - py
        dz = Z - pz
        d = dx * dx + dy * dy + dz * dz
        dists = jnp.minimum(dists, d)
        m = jnp.max(dists)
        fidx = jnp.min(jnp.where(dists == m, flat, 1e9))
        return dists, fidx

    dists0 = jnp.full(X.shape, 1e10, dtype=jnp.float32)
    jax.lax.fori_loop(0, MAX_INPUT, body, (dists0, 0.0))


def _fps_big(cx, feats):
    ins = [cx[:, 0].reshape(256, 128), cx[:, 1].reshape(256, 128),
           cx[:, 2].reshape(256, 128),
           feats[:, 0].reshape(256, 128), feats[:, 1].reshape(256, 128),
           feats[:, 2].reshape(256, 128)]
    out = pl.pallas_call(
        _fps_big_body,
        out_shape=jax.ShapeDtypeStruct((MAX_INPUT, 8), jnp.float32),
    )(*ins)
    return out


def _make_sa_body(P, Np, radius, nsample):
    r2 = radius * radius

    def body(kxyz_ref, kxyzT_ref, kfeats_ref, w1_ref, b1_ref, w2_ref, b2_ref,
             nxyz_ref, nfeat_ref):
        Xr = kxyzT_ref[0:1, :]
        Yr = kxyzT_ref[1:2, :]
        Zr = kxyzT_ref[2:3, :]
        lane = _iota((1, Np), 1)
        lane3 = _iota((1, 3), 1)

        def fps_body(i, carry):
            dists, fidx = carry
            oh = lane == fidx
            px = jnp.sum(jnp.where(oh, Xr, 0.0))
            py = jnp.sum(jnp.where(oh, Yr, 0.0))
            pz = jnp.sum(jnp.where(oh, Zr, 0.0))
            rv = (jnp.where(lane3 == 0.0, px, 0.0)
                  + jnp.where(lane3 == 1.0, py, 0.0)
                  + jnp.where(lane3 == 2.0, pz, 0.0))
            nxyz_ref[pl.ds(i, 1), :] = rv
            dx = Xr - px
            dy = Yr - py
            dz = Zr - pz
            d = dx * dx + dy * dy + dz * dz
            dists = jnp.minimum(dists, d)
            m = jnp.max(dists)
            fidx = jnp.min(jnp.where(dists == m, lane, 1e9))
            return dists, fidx

        dists0 = jnp.full((1, Np), 1e10, dtype=jnp.float32)
        jax.lax.fori_loop(0, P, fps_body, (dists0, 0.0))

        u = nxyz_ref[...]
        ux = u[:, 0:1]
        uy = u[:, 1:2]
        uz = u[:, 2:3]
        dx = ux - Xr
        dy = uy - Yr
        dz = uz - Zr
        d2 = dx * dx + dy * dy + dz * dz
        maskf = (d2 < r2).astype(jnp.float32)
        lt = (_iota((Np, Np), 0) < _iota((Np, Np), 1)).astype(jnp.float32)
        rank = _dot(maskf, lt, precision=jax.lax.Precision.DEFAULT)
        kcat = jnp.concatenate([kxyz_ref[...], kfeats_ref[...]], axis=1)

        run = None
        h0 = None
        for s in range(nsample):
            hs = maskf * (rank == float(s)).astype(jnp.float32)
            if s == 0:
                h0 = hs
            else:
                found = jnp.sum(hs, axis=1, keepdims=True)
                hs = jnp.where(found > 0.0, hs, h0)
            gs = _dot(hs, kcat)
            gs = jnp.concatenate([gs[:, 0:3] - u, gs[:, 3:]], axis=1)
            xs = jnp.maximum(_dot(gs, w1_ref[...]) + b1_ref[...], 0.0)
            xs = jnp.maximum(_dot(xs, w2_ref[...]) + b2_ref[...], 0.0)
            run = xs if run is None else jnp.maximum(run, xs)
        nfeat_ref[...] = run

    return body


def _sa(kxyz, kxyzT, kfeats, w1, b1, w2, b2, P, radius, nsample=16):
    Np = kxyz.shape[0]
    H2 = w2.shape[1]
    return pl.pallas_call(
        _make_sa_body(P, Np, radius, nsample),
        out_shape=(jax.ShapeDtypeStruct((P, 3), jnp.float32),
                   jax.ShapeDtypeStruct((P, H2), jnp.float32)),
    )(kxyz, kxyzT, kfeats, w1, b1, w2, b2)


def _make_fp_body(Nu, Nk):
    def body(uxyz_ref, kxyzT_ref, kfeats_ref, ufeats_ref,
             w1_ref, b1_ref, w2_ref, b2_ref, out_ref):
        ux = uxyz_ref[:, 0:1]
        uy = uxyz_ref[:, 1:2]
        uz = uxyz_ref[:, 2:3]
        Xr = kxyzT_ref[0:1, :]
        Yr = kxyzT_ref[1:2, :]
        Zr = kxyzT_ref[2:3, :]
        dx = ux - Xr
        dy = uy - Yr
        dz = uz - Zr
        d2 = dx * dx + dy * dy + dz * dz
        lane = _iota((1, Nk), 1)
        d2w = d2
        ohs, recips = [], []
        for _ in range(3):
            mj = jnp.min(d2w, axis=1, keepdims=True)
            idxj = jnp.min(jnp.where(d2w == mj, lane, 1e9),
                           axis=1, keepdims=True)
            oh = lane == idxj
            ohs.append(oh)
            recips.append(1.0 / (mj + 1e-8))
            d2w = jnp.where(oh, 1e30, d2w)
        ssum = (recips[0] + recips[1]) + recips[2]
        wm = (jnp.where(ohs[0], recips[0] / ssum, 0.0)
              + jnp.where(ohs[1], recips[1] / ssum, 0.0)
              + jnp.where(ohs[2], recips[2] / ssum, 0.0))
        interp = _dot(wm, kfeats_ref[...])
        x = jnp.concatenate([interp, ufeats_ref[...]], axis=1)
        x = jnp.maximum(_dot(x, w1_ref[...]) + b1_ref[...], 0.0)
        x = jnp.maximum(_dot(x, w2_ref[...]) + b2_ref[...], 0.0)
        out_ref[...] = x

    return body


def _fp(uxyz, kxyzT, kfeats, ufeats, w1, b1, w2, b2):
    Nu = uxyz.shape[0]
    Nk = kxyzT.shape[1]
    H2 = w2.shape[1]
    return pl.pallas_call(
        _make_fp_body(Nu, Nk),
        out_shape=jax.ShapeDtypeStruct((Nu, H2), jnp.float32),
    )(uxyz, kxyzT, kfeats, ufeats, w1, b1, w2, b2)


def _make_final_body(BLK, Nk):
    def body(cx_ref, kxyzT_ref, kfeats_ref, wf_ref, bf_ref, out_ref):
        ux = cx_ref[:, 0:1] * VOXEL
        uy = cx_ref[:, 1:2] * VOXEL
        uz = cx_ref[:, 2:3] * VOXEL
        Xr = kxyzT_ref[0:1, :]
        Yr = kxyzT_ref[1:2, :]
        Zr = kxyzT_ref[2:3, :]
        dx = ux - Xr
        dy = uy - Yr
        dz = uz - Zr
        d2 = dx * dx + dy * dy + dz * dz
        lane = _iota((1, Nk), 1)
        d2w = d2
        ohs, recips = [], []
        for _ in range(3):
            mj = jnp.min(d2w, axis=1, keepdims=True)
            idxj = jnp.min(jnp.where(d2w == mj, lane, 1e9),
                           axis=1, keepdims=True)
            oh = lane == idxj
            ohs.append(oh)
            recips.append(1.0 / (mj + 1e-8))
            d2w = jnp.where(oh, 1e30, d2w)
        ssum = (recips[0] + recips[1]) + recips[2]
        wm = (jnp.where(ohs[0], recips[0] / ssum, 0.0)
              + jnp.where(ohs[1], recips[1] / ssum, 0.0)
              + jnp.where(ohs[2], recips[2] / ssum, 0.0))
        interp = _dot(wm, kfeats_ref[...],
                      precision=jax.lax.Precision.DEFAULT)
        out_ref[...] = _dot(interp, wf_ref[...],
                            precision=jax.lax.Precision.DEFAULT) + bf_ref[...]

    return body


def _final(cx, kxyzT, kfeats, wf, bf):
    BLK = 2048
    N = cx.shape[0]
    Nk = kxyzT.shape[1]
    H = wf.shape[1]
    grid = N // BLK
    return pl.pallas_call(
        _make_final_body(BLK, Nk),
        grid=(grid,),
        in_specs=[
            pl.BlockSpec((BLK, 3), lambda i: (i, 0)),
            pl.BlockSpec((3, Nk), lambda i: (0, 0)),
            pl.BlockSpec((Nk, kfeats.shape[1]), lambda i: (0, 0)),
            pl.BlockSpec((wf.shape[0], H), lambda i: (0, 0)),
            pl.BlockSpec((1, H), lambda i: (0, 0)),
        ],
        out_specs=pl.BlockSpec((BLK, H), lambda i: (i, 0)),
        out_shape=jax.ShapeDtypeStruct((N, H), jnp.float32),
    )(cx, kxyzT, kfeats, wf, bf)


def _fold(layer):
    W, b, g, bt = layer
    s = g / jnp.sqrt(1.0 + BN_EPS)
    return (W * s[:, None]).T, (b * s + bt)[None, :]


def kernel(feats, coords, params):
    cx = coords[:, 1:4].astype(jnp.float32)
    sub = _fps_big(cx, feats)
    xyz_sub = sub[:, 0:3]
    feats_sub = sub[:, 3:6]
    sa1 = [p for l in params['sa1'] for p in _fold(l)]
    sa2 = [p for l in params['sa2'] for p in _fold(l)]
    sa3 = [p for l in params['sa3'] for p in _fold(l)]
    fp3 = [p for l in params['fp3'] for p in _fold(l)]
    fp2 = [p for l in params['fp2'] for p in _fold(l)]
    fp1 = [p for l in params['fp1'] for p in _fold(l)]
    wf, bf = _fold(params['final'])

    l1_xyz, l1_f = _sa(xyz_sub, xyz_sub.T, feats_sub, *sa1, P=256, radius=0.04)
    l2_xyz, l2_f = _sa(l1_xyz, l1_xyz.T, l1_f, *sa2, P=64, radius=0.08)
    l3_xyz, l3_f = _sa(l2_xyz, l2_xyz.T, l2_f, *sa3, P=16, radius=0.16)
    l2_f = _fp(l2_xyz, l3_xyz.T, l3_f, l2_f, *fp3)
    l1_f = _fp(l1_xyz, l2_xyz.T, l2_f, l1_f, *fp2)
    l0_f = _fp(xyz_sub, l1_xyz.T, l1_f, feats_sub, *fp1)
    return _final(cx, xyz_sub.T, l0_f, wf, bf)
